# baseline (speedup 1.0000x reference)
import functools

import jax
import jax.numpy as jnp
from jax import lax
from jax.experimental import pallas as pl
from jax.experimental.pallas import tpu as pltpu
from jax.experimental.pallas import tpu_sc as plsc

N = 307
U = 100
UP = 112
IN = 3
B = 64
E = 340
ET = E + N
NG = (ET + 15) // 16
ETP = NG * 16
BN = B * N
ROW_BLOCKS = 8
RB = BN // ROW_BLOCKS
NEG = -1e30
UPS = 113
NR = 312
SPAN = N * UPS
XSZ = BN * UPS
WIN = 34704


def _tc_pre_body(st_ref, ip_ref, lst_ref, lip_ref, lb_ref, rst_ref, rip_ref,
                 rb_ref, w1ip_ref, b1_ref, w2ip_ref, b2_ref,
                 xl_ref, xr_ref, p1_ref, p2_ref):
    st = st_ref[...]
    ip = ip_ref[...]
    dot = functools.partial(jnp.dot, preferred_element_type=jnp.float32)
    xl_ref[:, :UP] = dot(st, lst_ref[...]) + dot(ip, lip_ref[...]) + lb_ref[...]
    xr_ref[:, :UP] = dot(st, rst_ref[...]) + dot(ip, rip_ref[...]) + rb_ref[...]
    p1_ref[...] = dot(ip, w1ip_ref[...]) + b1_ref[...]
    p2_ref[...] = dot(ip, w2ip_ref[...]) + b2_ref[...]


def _tc_pre(st2d, ip2d, Lst, Lip, lbp, Rst, Rip, rbp, W1ip, b1p, W2ip, b2p):
    return pl.pallas_call(
        _tc_pre_body,
        grid=(ROW_BLOCKS,),
        in_specs=[
            pl.BlockSpec((RB, U), lambda i: (i, 0)),
            pl.BlockSpec((RB, IN), lambda i: (i, 0)),
            pl.BlockSpec((U, UP), lambda i: (0, 0)),
            pl.BlockSpec((IN, UP), lambda i: (0, 0)),
            pl.BlockSpec((1, UP), lambda i: (0, 0)),
            pl.BlockSpec((U, UP), lambda i: (0, 0)),
            pl.BlockSpec((IN, UP), lambda i: (0, 0)),
            pl.BlockSpec((1, UP), lambda i: (0, 0)),
            pl.BlockSpec((IN, 2 * UP), lambda i: (0, 0)),
            pl.BlockSpec((1, 2 * UP), lambda i: (0, 0)),
            pl.BlockSpec((IN, UP), lambda i: (0, 0)),
            pl.BlockSpec((1, UP), lambda i: (0, 0)),
        ],
        out_specs=[
            pl.BlockSpec((RB, UPS), lambda i: (i, 0)),
            pl.BlockSpec((RB, UPS), lambda i: (i, 0)),
            pl.BlockSpec((RB, 2 * UP), lambda i: (i, 0)),
            pl.BlockSpec((RB, UP), lambda i: (i, 0)),
        ],
        out_shape=[
            jax.ShapeDtypeStruct((BN, UPS), jnp.float32),
            jax.ShapeDtypeStruct((BN, UPS), jnp.float32),
            jax.ShapeDtypeStruct((BN, 2 * UP), jnp.float32),
            jax.ShapeDtypeStruct((BN, UP), jnp.float32),
        ],
    )(st2d, ip2d, Lst, Lip, lbp, Rst, Rip, rbp, W1ip, b1p, W2ip, b2p)


def _tc_post_body(x1_ref, p1_ref, p2_ref, w1h_ref, w2h_ref, bv_ref, out_ref):
    dot = functools.partial(jnp.dot, preferred_element_type=jnp.float32)
    x1p = x1_ref[...]
    x1 = jnp.concatenate(
        [x1p[k * NR:k * NR + N, :UP] for k in range(B // ROW_BLOCKS)], axis=0)
    st2 = x1 + bv_ref[...]
    v = jax.nn.sigmoid(p1_ref[...] + dot(st2, w1h_ref[...]))
    r = v[:, :UP]
    u = v[:, UP:]
    c = jnp.tanh(p2_ref[...] + dot(r * st2, w2h_ref[...]))
    o = u * st2 + (1.0 - u) * c
    out_ref[...] = o[:, :U]


def _tc_post(x1, P1, P2, W1h, W2h, bvec):
    return pl.pallas_call(
        _tc_post_body,
        grid=(ROW_BLOCKS,),
        in_specs=[
            pl.BlockSpec((B // ROW_BLOCKS * NR, UPS), lambda i: (i, 0)),
            pl.BlockSpec((RB, 2 * UP), lambda i: (i, 0)),
            pl.BlockSpec((RB, UP), lambda i: (i, 0)),
            pl.BlockSpec((UP, 2 * UP), lambda i: (0, 0)),
            pl.BlockSpec((UP, UP), lambda i: (0, 0)),
            pl.BlockSpec((1, UP), lambda i: (0, 0)),
        ],
        out_specs=pl.BlockSpec((RB, U), lambda i: (i, 0)),
        out_shape=jax.ShapeDtypeStruct((BN, U), jnp.float32),
    )(x1, P1, P2, W1h, W2h, bvec)


_info = plsc.get_sparse_core_info()
_NC = _info.num_cores
_NS = _info.num_subcores
_NW = _NC * _NS
_BPW = B // _NW
_DEN = 320


def _sc_edge_body(xl_hbm, xr_hbm, src_hbm, dst_hbm, att_hbm, out_hbm,
                  xl_v, xr_v, out_v, src_v, dst_v, att_v, logit_v, e_v,
                  denom_v):
    cid = lax.axis_index("c")
    sid = lax.axis_index("s")
    wid = sid * _NC + cid
    pltpu.sync_copy(src_hbm, src_v)
    pltpu.sync_copy(dst_hbm, dst_v)
    pltpu.sync_copy(att_hbm, att_v)
    zero16 = jnp.zeros((16,), jnp.float32)

    for bi in range(_BPW):
        b = wid * _BPW + bi
        off = jnp.minimum((b * SPAN) // 8 * 8, XSZ - WIN)
        off = pl.multiple_of(off, 8)
        delta = b * SPAN - off
        pltpu.sync_copy(xl_hbm.at[pl.ds(off, WIN)], xl_v)
        pltpu.sync_copy(xr_hbm.at[pl.ds(off, WIN)], xr_v)

        def group_a(g, gmax):
            sbase = src_v[pl.ds(g * 16, 16)] * UPS + delta
            dbase = dst_v[pl.ds(g * 16, 16)] * UPS + delta

            def chunk_a(cu, accs):
                attc = att_v[pl.ds(cu * 16, 16)]
                bs = sbase + cu * 16
                bd = dbase + cu * 16
                outs = list(accs)
                for j in range(16):
                    xlc = plsc.load_gather(xl_v, [bs + j])
                    xrc = plsc.load_gather(xr_v, [bd + j])
                    m = xlc + xrc
                    m = jnp.maximum(m, 0.2 * m)
                    outs[j % 4] = outs[j % 4] + m * attc[j]
                return tuple(outs)

            a0, a1, a2, a3 = lax.fori_loop(0, UP // 16, chunk_a, (zero16,) * 4)
            acc = (a0 + a1) + (a2 + a3)
            lane = g * 16 + lax.iota(jnp.int32, 16)
            lg = jnp.where(lane < ET, acc, NEG)
            logit_v[pl.ds(g * 16, 16)] = lg
            return jnp.maximum(gmax, lg)

        gmaxv = lax.fori_loop(0, NG, group_a, jnp.full((16,), NEG, jnp.float32))
        gmax = jnp.max(gmaxv)

        for i in range(_DEN // 16):
            denom_v[pl.ds(i * 16, 16)] = zero16

        def group_b(g, carry):
            lg = logit_v[pl.ds(g * 16, 16)]
            e16 = jnp.exp(lg - gmax)
            e_v[pl.ds(g * 16, 16)] = e16
            dst16 = dst_v[pl.ds(g * 16, 16)]
            plsc.addupdate_scatter(denom_v, [dst16], e16)
            return carry

        lax.fori_loop(0, NG, group_b, 0)

        def zout(i, carry):
            for j in range(UP // 16):
                out_v[i, pl.ds(j * 16, 16)] = zero16
            return carry

        lax.fori_loop(0, N, zout, 0)

        def group_c(g, carry):
            dst16 = dst_v[pl.ds(g * 16, 16)]
            sbase = src_v[pl.ds(g * 16, 16)] * UPS + delta
            e16 = e_v[pl.ds(g * 16, 16)]
            den = plsc.load_gather(denom_v, [dst16])
            alpha = e16 / (den + 1e-16)

            def chunk_c(cu, carry2):
                bs = sbase + cu * 16
                c0 = cu * 16
                for j in range(16):
                    xlc = plsc.load_gather(xl_v, [bs + j])
                    cj = jnp.full((16,), c0 + j, jnp.int32)
                    plsc.addupdate_scatter(out_v, [dst16, cj], alpha * xlc)
                return carry2

            lax.fori_loop(0, UP // 16, chunk_c, 0)
            return carry

        lax.fori_loop(0, NG, group_c, 0)
        pltpu.sync_copy(out_v, out_hbm.at[pl.ds(pl.multiple_of(b * NR, 8), NR)])


_sc_edge = functools.partial(
    pl.kernel,
    out_type=jax.ShapeDtypeStruct((B * NR, UPS), jnp.float32),
    mesh=plsc.VectorSubcoreMesh(core_axis_name="c", subcore_axis_name="s"),
    compiler_params=pltpu.CompilerParams(needs_layout_passes=False),
    scratch_types=[
        pltpu.VMEM((WIN,), jnp.float32),
        pltpu.VMEM((WIN,), jnp.float32),
        pltpu.VMEM((NR, UPS), jnp.float32),
        pltpu.VMEM((ETP,), jnp.int32),
        pltpu.VMEM((ETP,), jnp.int32),
        pltpu.VMEM((UP + 16,), jnp.float32),
        pltpu.VMEM((ETP,), jnp.float32),
        pltpu.VMEM((ETP,), jnp.float32),
        pltpu.VMEM((_DEN,), jnp.float32),
    ],
)(_sc_edge_body)


def kernel(inputs, state, edge_index, bias_1, W_gcn1, b_gcn1, W_gcn2, b_gcn2,
           linl_w, linl_b, linr_w, linr_b, att, gat_bias):
    ip2d = inputs.reshape(BN, IN)
    st2d = state.reshape(BN, U)
    loops = jnp.arange(N, dtype=edge_index.dtype)
    src = jnp.pad(jnp.concatenate([edge_index[0], loops]), (0, ETP - ET))
    dst = jnp.pad(jnp.concatenate([edge_index[1], loops]), (0, ETP - ET))

    pad1 = lambda v: jnp.pad(v, (0, UP - U))
    row1 = lambda v: v.reshape(1, -1)
    Lst = jnp.pad(linl_w[:U], [(0, 0), (0, UP - U)])
    Lip = jnp.pad(linl_w[U:], [(0, 0), (0, UP - U)])
    Rst = jnp.pad(linr_w[:U], [(0, 0), (0, UP - U)])
    Rip = jnp.pad(linr_w[U:], [(0, 0), (0, UP - U)])
    W1h = jnp.concatenate(
        [jnp.pad(W_gcn1[IN:, :U], [(0, UP - U), (0, UP - U)]),
         jnp.pad(W_gcn1[IN:, U:], [(0, UP - U), (0, UP - U)])], axis=1)
    W1ip = jnp.concatenate(
        [jnp.pad(W_gcn1[:IN, :U], [(0, 0), (0, UP - U)]),
         jnp.pad(W_gcn1[:IN, U:], [(0, 0), (0, UP - U)])], axis=1)
    b1p = jnp.concatenate([pad1(b_gcn1[:U]), pad1(b_gcn1[U:])])
    W2h = jnp.pad(W_gcn2[IN:], [(0, UP - U), (0, UP - U)])
    W2ip = jnp.pad(W_gcn2[:IN], [(0, 0), (0, UP - U)])

    XL, XR, P1, P2 = _tc_pre(st2d, ip2d, Lst, Lip, row1(pad1(linl_b)),
                             Rst, Rip, row1(pad1(linr_b)),
                             W1ip, row1(b1p), W2ip, row1(pad1(b_gcn2)))

    x1 = _sc_edge(XL.reshape(XSZ), XR.reshape(XSZ),
                  src.astype(jnp.int32), dst.astype(jnp.int32),
                  jnp.pad(att, (0, UP + 16 - U)))

    out = _tc_post(x1, P1, P2, W1h, W2h,
                   row1(pad1(bias_1 + gat_bias)))
    return out.reshape(B, N * U)


# speedup vs baseline: 1.1984x; 1.1984x over previous
import functools

import jax
import jax.numpy as jnp
from jax import lax
from jax.experimental import pallas as pl
from jax.experimental.pallas import tpu as pltpu
from jax.experimental.pallas import tpu_sc as plsc

N = 307
U = 100
UP = 112
IN = 3
B = 64
E = 340
ET = E + N
NG = (ET + 15) // 16
ETP = NG * 16
BN = B * N
ROW_BLOCKS = 8
RB = BN // ROW_BLOCKS
NEG = -1e30
UPS = 113
NR = 312
SPAN = N * UPS
XSZ = BN * UPS
WIN = 34704


def _tc_pre_body(st_ref, ip_ref, lst_ref, lip_ref, lb_ref, rst_ref, rip_ref,
                 rb_ref, w1ip_ref, b1_ref, w2ip_ref, b2_ref,
                 xl_ref, xr_ref, p1_ref, p2_ref):
    st = st_ref[...]
    ip = ip_ref[...]
    dot = functools.partial(jnp.dot, preferred_element_type=jnp.float32)
    xl_ref[:, :UP] = dot(st, lst_ref[...]) + dot(ip, lip_ref[...]) + lb_ref[...]
    xr_ref[:, :UP] = dot(st, rst_ref[...]) + dot(ip, rip_ref[...]) + rb_ref[...]
    p1_ref[...] = dot(ip, w1ip_ref[...]) + b1_ref[...]
    p2_ref[...] = dot(ip, w2ip_ref[...]) + b2_ref[...]


def _tc_pre(st2d, ip2d, Lst, Lip, lbp, Rst, Rip, rbp, W1ip, b1p, W2ip, b2p):
    return pl.pallas_call(
        _tc_pre_body,
        grid=(ROW_BLOCKS,),
        in_specs=[
            pl.BlockSpec((RB, U), lambda i: (i, 0)),
            pl.BlockSpec((RB, IN), lambda i: (i, 0)),
            pl.BlockSpec((U, UP), lambda i: (0, 0)),
            pl.BlockSpec((IN, UP), lambda i: (0, 0)),
            pl.BlockSpec((1, UP), lambda i: (0, 0)),
            pl.BlockSpec((U, UP), lambda i: (0, 0)),
            pl.BlockSpec((IN, UP), lambda i: (0, 0)),
            pl.BlockSpec((1, UP), lambda i: (0, 0)),
            pl.BlockSpec((IN, 2 * UP), lambda i: (0, 0)),
            pl.BlockSpec((1, 2 * UP), lambda i: (0, 0)),
            pl.BlockSpec((IN, UP), lambda i: (0, 0)),
            pl.BlockSpec((1, UP), lambda i: (0, 0)),
        ],
        out_specs=[
            pl.BlockSpec((RB, UPS), lambda i: (i, 0)),
            pl.BlockSpec((RB, UPS), lambda i: (i, 0)),
            pl.BlockSpec((RB, 2 * UP), lambda i: (i, 0)),
            pl.BlockSpec((RB, UP), lambda i: (i, 0)),
        ],
        out_shape=[
            jax.ShapeDtypeStruct((BN, UPS), jnp.float32),
            jax.ShapeDtypeStruct((BN, UPS), jnp.float32),
            jax.ShapeDtypeStruct((BN, 2 * UP), jnp.float32),
            jax.ShapeDtypeStruct((BN, UP), jnp.float32),
        ],
    )(st2d, ip2d, Lst, Lip, lbp, Rst, Rip, rbp, W1ip, b1p, W2ip, b2p)


def _tc_post_body(x1_ref, p1_ref, p2_ref, w1h_ref, w2h_ref, bv_ref, out_ref):
    dot = functools.partial(jnp.dot, preferred_element_type=jnp.float32)
    st2 = x1_ref[...] + bv_ref[...]
    v = jax.nn.sigmoid(p1_ref[...] + dot(st2, w1h_ref[...]))
    r = v[:, :UP]
    u = v[:, UP:]
    c = jnp.tanh(p2_ref[...] + dot(r * st2, w2h_ref[...]))
    o = u * st2 + (1.0 - u) * c
    out_ref[...] = o[:, :U]


def _tc_post(x1, P1, P2, W1h, W2h, bvec):
    return pl.pallas_call(
        _tc_post_body,
        grid=(ROW_BLOCKS,),
        in_specs=[
            pl.BlockSpec((RB, UP), lambda i: (i, 0)),
            pl.BlockSpec((RB, 2 * UP), lambda i: (i, 0)),
            pl.BlockSpec((RB, UP), lambda i: (i, 0)),
            pl.BlockSpec((UP, 2 * UP), lambda i: (0, 0)),
            pl.BlockSpec((UP, UP), lambda i: (0, 0)),
            pl.BlockSpec((1, UP), lambda i: (0, 0)),
        ],
        out_specs=pl.BlockSpec((RB, U), lambda i: (i, 0)),
        out_shape=jax.ShapeDtypeStruct((BN, U), jnp.float32),
    )(x1, P1, P2, W1h, W2h, bvec)


_info = plsc.get_sparse_core_info()
_NC = _info.num_cores
_NS = _info.num_subcores
_NW = _NC * _NS
_BPW = B // _NW
_DEN = 320


def _sc_edge_body(xl_hbm, xr_hbm, src_hbm, dst_hbm, att_hbm, out_hbm,
                  xl_v, xr_v, out_v, src_v, dst_v, att_v, logit_v, e_v,
                  denom_v):
    cid = lax.axis_index("c")
    sid = lax.axis_index("s")
    wid = sid * _NC + cid
    pltpu.sync_copy(src_hbm, src_v)
    pltpu.sync_copy(dst_hbm, dst_v)
    pltpu.sync_copy(att_hbm, att_v)
    zero16 = jnp.zeros((16,), jnp.float32)

    for bi in range(_BPW):
        b = wid * _BPW + bi
        off = jnp.minimum((b * SPAN) // 8 * 8, XSZ - WIN)
        off = pl.multiple_of(off, 8)
        delta = b * SPAN - off
        pltpu.sync_copy(xl_hbm.at[pl.ds(off, WIN)], xl_v)
        pltpu.sync_copy(xr_hbm.at[pl.ds(off, WIN)], xr_v)

        def group_a(g, gmax):
            sbase = src_v[pl.ds(g * 16, 16)] * UPS + delta
            dbase = dst_v[pl.ds(g * 16, 16)] * UPS + delta

            def chunk_a(cu, accs):
                attc = att_v[pl.ds(cu * 16, 16)]
                bs = sbase + cu * 16
                bd = dbase + cu * 16
                outs = list(accs)
                for j in range(16):
                    xlc = plsc.load_gather(xl_v, [bs + j])
                    xrc = plsc.load_gather(xr_v, [bd + j])
                    m = xlc + xrc
                    m = jnp.maximum(m, 0.2 * m)
                    outs[j % 4] = outs[j % 4] + m * attc[j]
                return tuple(outs)

            a0, a1, a2, a3 = lax.fori_loop(0, UP // 16, chunk_a, (zero16,) * 4)
            acc = (a0 + a1) + (a2 + a3)
            lane = g * 16 + lax.iota(jnp.int32, 16)
            lg = jnp.where(lane < ET, acc, NEG)
            logit_v[pl.ds(g * 16, 16)] = lg
            return jnp.maximum(gmax, lg)

        gmaxv = lax.fori_loop(0, NG, group_a, jnp.full((16,), NEG, jnp.float32))
        gmax = jnp.max(gmaxv)

        for i in range(_DEN // 16):
            denom_v[pl.ds(i * 16, 16)] = zero16

        def group_b(g, carry):
            lg = logit_v[pl.ds(g * 16, 16)]
            e16 = jnp.exp(lg - gmax)
            e_v[pl.ds(g * 16, 16)] = e16
            dst16 = dst_v[pl.ds(g * 16, 16)]
            plsc.addupdate_scatter(denom_v, [dst16], e16)
            return carry

        lax.fori_loop(0, NG, group_b, 0)

        def zout(i, carry):
            for j in range(4):
                out_v[pl.ds((i * 4 + j) * 16, 16)] = zero16
            return carry

        lax.fori_loop(0, 544, zout, 0)

        def group_c(g, carry):
            dst16 = dst_v[pl.ds(g * 16, 16)]
            sbase = src_v[pl.ds(g * 16, 16)] * UPS + delta
            dbase = dst16 * UPS
            e16 = e_v[pl.ds(g * 16, 16)]
            den = plsc.load_gather(denom_v, [dst16])
            alpha = e16 / (den + 1e-16)

            def chunk_c(cu, carry2):
                bs = sbase + cu * 16
                bd = dbase + cu * 16
                for j in range(16):
                    xlc = plsc.load_gather(xl_v, [bs + j])
                    plsc.addupdate_scatter(out_v, [bd + j], alpha * xlc)
                return carry2

            lax.fori_loop(0, UP // 16, chunk_c, 0)
            return carry

        lax.fori_loop(0, NG, group_c, 0)
        pltpu.sync_copy(out_v, out_hbm.at[b])


_sc_edge = functools.partial(
    pl.kernel,
    out_type=jax.ShapeDtypeStruct((B, NR * UPS), jnp.float32),
    mesh=plsc.VectorSubcoreMesh(core_axis_name="c", subcore_axis_name="s"),
    compiler_params=pltpu.CompilerParams(needs_layout_passes=False),
    scratch_types=[
        pltpu.VMEM((WIN,), jnp.float32),
        pltpu.VMEM((WIN,), jnp.float32),
        pltpu.VMEM((NR * UPS,), jnp.float32),
        pltpu.VMEM((ETP,), jnp.int32),
        pltpu.VMEM((ETP,), jnp.int32),
        pltpu.VMEM((UP + 16,), jnp.float32),
        pltpu.VMEM((ETP,), jnp.float32),
        pltpu.VMEM((ETP,), jnp.float32),
        pltpu.VMEM((_DEN,), jnp.float32),
    ],
)(_sc_edge_body)


def kernel(inputs, state, edge_index, bias_1, W_gcn1, b_gcn1, W_gcn2, b_gcn2,
           linl_w, linl_b, linr_w, linr_b, att, gat_bias):
    ip2d = inputs.reshape(BN, IN)
    st2d = state.reshape(BN, U)
    loops = jnp.arange(N, dtype=edge_index.dtype)
    src = jnp.pad(jnp.concatenate([edge_index[0], loops]), (0, ETP - ET))
    dst = jnp.pad(jnp.concatenate([edge_index[1], loops]), (0, ETP - ET))

    pad1 = lambda v: jnp.pad(v, (0, UP - U))
    row1 = lambda v: v.reshape(1, -1)
    Lst = jnp.pad(linl_w[:U], [(0, 0), (0, UP - U)])
    Lip = jnp.pad(linl_w[U:], [(0, 0), (0, UP - U)])
    Rst = jnp.pad(linr_w[:U], [(0, 0), (0, UP - U)])
    Rip = jnp.pad(linr_w[U:], [(0, 0), (0, UP - U)])
    W1h = jnp.concatenate(
        [jnp.pad(W_gcn1[IN:, :U], [(0, UP - U), (0, UP - U)]),
         jnp.pad(W_gcn1[IN:, U:], [(0, UP - U), (0, UP - U)])], axis=1)
    W1ip = jnp.concatenate(
        [jnp.pad(W_gcn1[:IN, :U], [(0, 0), (0, UP - U)]),
         jnp.pad(W_gcn1[:IN, U:], [(0, 0), (0, UP - U)])], axis=1)
    b1p = jnp.concatenate([pad1(b_gcn1[:U]), pad1(b_gcn1[U:])])
    W2h = jnp.pad(W_gcn2[IN:], [(0, UP - U), (0, UP - U)])
    W2ip = jnp.pad(W_gcn2[:IN], [(0, 0), (0, UP - U)])

    XL, XR, P1, P2 = _tc_pre(st2d, ip2d, Lst, Lip, row1(pad1(linl_b)),
                             Rst, Rip, row1(pad1(linr_b)),
                             W1ip, row1(b1p), W2ip, row1(pad1(b_gcn2)))

    x1 = _sc_edge(XL.reshape(XSZ), XR.reshape(XSZ),
                  src.astype(jnp.int32), dst.astype(jnp.int32),
                  jnp.pad(att, (0, UP + 16 - U)))

    x1u = x1.reshape(B, NR, UPS)[:, :N, :UP].reshape(BN, UP)
    out = _tc_post(x1u, P1, P2, W1h, W2h,
                   row1(pad1(bias_1 + gat_bias)))
    return out.reshape(B, N * U)


# speedup vs baseline: 1.2258x; 1.0228x over previous
import functools

import jax
import jax.numpy as jnp
from jax import lax
from jax.experimental import pallas as pl
from jax.experimental.pallas import tpu as pltpu
from jax.experimental.pallas import tpu_sc as plsc

N = 307
U = 100
UP = 112
IN = 3
B = 64
E = 340
ET = E + N
NG = (ET + 15) // 16
ETP = NG * 16
BN = B * N
ROW_BLOCKS = 8
RB = BN // ROW_BLOCKS
NEG = -1e30
UPS = 113
NR = 312
UPW = 227
XRO = 112
SPAN = N * UPW
XSZ = BN * UPW
WIN = 69712


def _tc_pre_body(st_ref, ip_ref, lst_ref, lip_ref, lb_ref, rst_ref, rip_ref,
                 rb_ref, w1ip_ref, b1_ref, w2ip_ref, b2_ref,
                 xlr_ref, p1_ref, p2_ref):
    st = st_ref[...]
    ip = ip_ref[...]
    dot = functools.partial(jnp.dot, preferred_element_type=jnp.float32)
    xlr_ref[:, :UP] = dot(st, lst_ref[...]) + dot(ip, lip_ref[...]) + lb_ref[...]
    xlr_ref[:, XRO:XRO + UP] = (dot(st, rst_ref[...]) + dot(ip, rip_ref[...])
                                + rb_ref[...])
    p1_ref[...] = dot(ip, w1ip_ref[...]) + b1_ref[...]
    p2_ref[...] = dot(ip, w2ip_ref[...]) + b2_ref[...]


def _tc_pre(st2d, ip2d, Lst, Lip, lbp, Rst, Rip, rbp, W1ip, b1p, W2ip, b2p):
    return pl.pallas_call(
        _tc_pre_body,
        grid=(ROW_BLOCKS,),
        in_specs=[
            pl.BlockSpec((RB, U), lambda i: (i, 0)),
            pl.BlockSpec((RB, IN), lambda i: (i, 0)),
            pl.BlockSpec((U, UP), lambda i: (0, 0)),
            pl.BlockSpec((IN, UP), lambda i: (0, 0)),
            pl.BlockSpec((1, UP), lambda i: (0, 0)),
            pl.BlockSpec((U, UP), lambda i: (0, 0)),
            pl.BlockSpec((IN, UP), lambda i: (0, 0)),
            pl.BlockSpec((1, UP), lambda i: (0, 0)),
            pl.BlockSpec((IN, 2 * UP), lambda i: (0, 0)),
            pl.BlockSpec((1, 2 * UP), lambda i: (0, 0)),
            pl.BlockSpec((IN, UP), lambda i: (0, 0)),
            pl.BlockSpec((1, UP), lambda i: (0, 0)),
        ],
        out_specs=[
            pl.BlockSpec((RB, UPW), lambda i: (i, 0)),
            pl.BlockSpec((RB, 2 * UP), lambda i: (i, 0)),
            pl.BlockSpec((RB, UP), lambda i: (i, 0)),
        ],
        out_shape=[
            jax.ShapeDtypeStruct((BN, UPW), jnp.float32),
            jax.ShapeDtypeStruct((BN, 2 * UP), jnp.float32),
            jax.ShapeDtypeStruct((BN, UP), jnp.float32),
        ],
    )(st2d, ip2d, Lst, Lip, lbp, Rst, Rip, rbp, W1ip, b1p, W2ip, b2p)


def _tc_post_body(x1_ref, p1_ref, p2_ref, w1h_ref, w2h_ref, bv_ref, out_ref):
    dot = functools.partial(jnp.dot, preferred_element_type=jnp.float32)
    st2 = x1_ref[...] + bv_ref[...]
    v = jax.nn.sigmoid(p1_ref[...] + dot(st2, w1h_ref[...]))
    r = v[:, :UP]
    u = v[:, UP:]
    c = jnp.tanh(p2_ref[...] + dot(r * st2, w2h_ref[...]))
    o = u * st2 + (1.0 - u) * c
    out_ref[...] = o[:, :U]


def _tc_post(x1, P1, P2, W1h, W2h, bvec):
    return pl.pallas_call(
        _tc_post_body,
        grid=(ROW_BLOCKS,),
        in_specs=[
            pl.BlockSpec((RB, UP), lambda i: (i, 0)),
            pl.BlockSpec((RB, 2 * UP), lambda i: (i, 0)),
            pl.BlockSpec((RB, UP), lambda i: (i, 0)),
            pl.BlockSpec((UP, 2 * UP), lambda i: (0, 0)),
            pl.BlockSpec((UP, UP), lambda i: (0, 0)),
            pl.BlockSpec((1, UP), lambda i: (0, 0)),
        ],
        out_specs=pl.BlockSpec((RB, U), lambda i: (i, 0)),
        out_shape=jax.ShapeDtypeStruct((BN, U), jnp.float32),
    )(x1, P1, P2, W1h, W2h, bvec)


_info = plsc.get_sparse_core_info()
_NC = _info.num_cores
_NS = _info.num_subcores
_NW = _NC * _NS
_BPW = B // _NW
_DEN = 320


def _sc_edge_body(xlr_hbm, src_hbm, dst_hbm, att_hbm, out_hbm,
                  xlr_v, out_v, src_v, dst_v, att_v, logit_v, e_v,
                  denom_v):
    cid = lax.axis_index("c")
    sid = lax.axis_index("s")
    wid = sid * _NC + cid
    pltpu.sync_copy(src_hbm, src_v)
    pltpu.sync_copy(dst_hbm, dst_v)
    pltpu.sync_copy(att_hbm, att_v)
    zero16 = jnp.zeros((16,), jnp.float32)

    for bi in range(_BPW):
        b = wid * _BPW + bi
        off = jnp.minimum((b * SPAN) // 8 * 8, XSZ - WIN)
        off = pl.multiple_of(off, 8)
        delta = b * SPAN - off
        pltpu.sync_copy(xlr_hbm.at[pl.ds(off, WIN)], xlr_v)

        def group_a(g, gmax):
            sbase = src_v[pl.ds(g * 16, 16)] * UPW + delta
            dbase = dst_v[pl.ds(g * 16, 16)] * UPW + (delta + XRO)

            def chunk_a(cu, accs):
                attc = att_v[pl.ds(cu * 16, 16)]
                bs = sbase + cu * 16
                bd = dbase + cu * 16
                outs = list(accs)
                for j in range(16):
                    xlc = plsc.load_gather(xlr_v, [bs + j])
                    xrc = plsc.load_gather(xlr_v, [bd + j])
                    m = xlc + xrc
                    m = jnp.maximum(m, 0.2 * m)
                    outs[j % 4] = outs[j % 4] + m * attc[j]
                return tuple(outs)

            a0, a1, a2, a3 = lax.fori_loop(0, UP // 16, chunk_a, (zero16,) * 4)
            acc = (a0 + a1) + (a2 + a3)
            lane = g * 16 + lax.iota(jnp.int32, 16)
            lg = jnp.where(lane < ET, acc, NEG)
            logit_v[pl.ds(g * 16, 16)] = lg
            return jnp.maximum(gmax, lg)

        gmaxv = lax.fori_loop(0, NG, group_a, jnp.full((16,), NEG, jnp.float32))
        gmax = jnp.max(gmaxv)

        for i in range(_DEN // 16):
            denom_v[pl.ds(i * 16, 16)] = zero16

        def group_b(g, carry):
            lg = logit_v[pl.ds(g * 16, 16)]
            e16 = jnp.exp(lg - gmax)
            e_v[pl.ds(g * 16, 16)] = e16
            dst16 = dst_v[pl.ds(g * 16, 16)]
            plsc.addupdate_scatter(denom_v, [dst16], e16)
            return carry

        lax.fori_loop(0, NG, group_b, 0)

        def zout(i, carry):
            for j in range(4):
                out_v[pl.ds((i * 4 + j) * 16, 16)] = zero16
            return carry

        lax.fori_loop(0, 544, zout, 0)

        def group_c(g, carry):
            dst16 = dst_v[pl.ds(g * 16, 16)]
            sbase = src_v[pl.ds(g * 16, 16)] * UPW + delta
            dbase = dst16 * UPS
            e16 = e_v[pl.ds(g * 16, 16)]
            den = plsc.load_gather(denom_v, [dst16])
            alpha = e16 / (den + 1e-16)

            def chunk_c(cu, carry2):
                bs = sbase + cu * 16
                bd = dbase + cu * 16
                for j in range(16):
                    xlc = plsc.load_gather(xlr_v, [bs + j])
                    plsc.addupdate_scatter(out_v, [bd + j], alpha * xlc)
                return carry2

            lax.fori_loop(0, UP // 16, chunk_c, 0)
            return carry

        lax.fori_loop(0, NG, group_c, 0)
        pltpu.sync_copy(out_v, out_hbm.at[b])


_sc_edge = functools.partial(
    pl.kernel,
    out_type=jax.ShapeDtypeStruct((B, NR * UPS), jnp.float32),
    mesh=plsc.VectorSubcoreMesh(core_axis_name="c", subcore_axis_name="s"),
    compiler_params=pltpu.CompilerParams(needs_layout_passes=False),
    scratch_types=[
        pltpu.VMEM((WIN,), jnp.float32),
        pltpu.VMEM((NR * UPS,), jnp.float32),
        pltpu.VMEM((ETP,), jnp.int32),
        pltpu.VMEM((ETP,), jnp.int32),
        pltpu.VMEM((UP + 16,), jnp.float32),
        pltpu.VMEM((ETP,), jnp.float32),
        pltpu.VMEM((ETP,), jnp.float32),
        pltpu.VMEM((_DEN,), jnp.float32),
    ],
)(_sc_edge_body)


def kernel(inputs, state, edge_index, bias_1, W_gcn1, b_gcn1, W_gcn2, b_gcn2,
           linl_w, linl_b, linr_w, linr_b, att, gat_bias):
    ip2d = inputs.reshape(BN, IN)
    st2d = state.reshape(BN, U)
    loops = jnp.arange(N, dtype=edge_index.dtype)
    src = jnp.pad(jnp.concatenate([edge_index[0], loops]), (0, ETP - ET))
    dst = jnp.pad(jnp.concatenate([edge_index[1], loops]), (0, ETP - ET))

    pad1 = lambda v: jnp.pad(v, (0, UP - U))
    row1 = lambda v: v.reshape(1, -1)
    Lst = jnp.pad(linl_w[:U], [(0, 0), (0, UP - U)])
    Lip = jnp.pad(linl_w[U:], [(0, 0), (0, UP - U)])
    Rst = jnp.pad(linr_w[:U], [(0, 0), (0, UP - U)])
    Rip = jnp.pad(linr_w[U:], [(0, 0), (0, UP - U)])
    W1h = jnp.concatenate(
        [jnp.pad(W_gcn1[IN:, :U], [(0, UP - U), (0, UP - U)]),
         jnp.pad(W_gcn1[IN:, U:], [(0, UP - U), (0, UP - U)])], axis=1)
    W1ip = jnp.concatenate(
        [jnp.pad(W_gcn1[:IN, :U], [(0, 0), (0, UP - U)]),
         jnp.pad(W_gcn1[:IN, U:], [(0, 0), (0, UP - U)])], axis=1)
    b1p = jnp.concatenate([pad1(b_gcn1[:U]), pad1(b_gcn1[U:])])
    W2h = jnp.pad(W_gcn2[IN:], [(0, UP - U), (0, UP - U)])
    W2ip = jnp.pad(W_gcn2[:IN], [(0, 0), (0, UP - U)])

    XLR, P1, P2 = _tc_pre(st2d, ip2d, Lst, Lip, row1(pad1(linl_b)),
                          Rst, Rip, row1(pad1(linr_b)),
                          W1ip, row1(b1p), W2ip, row1(pad1(b_gcn2)))

    x1 = _sc_edge(XLR.reshape(XSZ),
                  src.astype(jnp.int32), dst.astype(jnp.int32),
                  jnp.pad(att, (0, UP + 16 - U)))

    x1u = x1.reshape(B, NR, UPS)[:, :N, :UP].reshape(BN, UP)
    out = _tc_post(x1u, P1, P2, W1h, W2h,
                   row1(pad1(bias_1 + gat_bias)))
    return out.reshape(B, N * U)


# speedup vs baseline: 1.2521x; 1.0215x over previous
import functools

import jax
import jax.numpy as jnp
from jax import lax
from jax.experimental import pallas as pl
from jax.experimental.pallas import tpu as pltpu
from jax.experimental.pallas import tpu_sc as plsc

N = 307
U = 100
UP = 112
IN = 3
B = 64
E = 340
ET = E + N
NG = (ET + 15) // 16
ETP = NG * 16
BN = B * N
ROW_BLOCKS = 8
RB = BN // ROW_BLOCKS
NEG = -1e30
UPS = 113
NR = 312
UPW = 227
XRO = 112
SPAN = N * UPW
XSZ = BN * UPW
WIN = 69712


def _tc_pre_body(st_ref, ip_ref, lst_ref, lip_ref, lb_ref, rst_ref, rip_ref,
                 rb_ref, w1ip_ref, b1_ref, w2ip_ref, b2_ref,
                 xlr_ref, p1_ref, p2_ref):
    st = st_ref[...]
    ip = ip_ref[...]
    dot = functools.partial(jnp.dot, preferred_element_type=jnp.float32)
    xlr_ref[:, :UP] = dot(st, lst_ref[...]) + dot(ip, lip_ref[...]) + lb_ref[...]
    xlr_ref[:, XRO:XRO + UP] = (dot(st, rst_ref[...]) + dot(ip, rip_ref[...])
                                + rb_ref[...])
    p1_ref[...] = dot(ip, w1ip_ref[...]) + b1_ref[...]
    p2_ref[...] = dot(ip, w2ip_ref[...]) + b2_ref[...]


def _tc_pre(st2d, ip2d, Lst, Lip, lbp, Rst, Rip, rbp, W1ip, b1p, W2ip, b2p):
    return pl.pallas_call(
        _tc_pre_body,
        grid=(ROW_BLOCKS,),
        in_specs=[
            pl.BlockSpec((RB, U), lambda i: (i, 0)),
            pl.BlockSpec((RB, IN), lambda i: (i, 0)),
            pl.BlockSpec((U, UP), lambda i: (0, 0)),
            pl.BlockSpec((IN, UP), lambda i: (0, 0)),
            pl.BlockSpec((1, UP), lambda i: (0, 0)),
            pl.BlockSpec((U, UP), lambda i: (0, 0)),
            pl.BlockSpec((IN, UP), lambda i: (0, 0)),
            pl.BlockSpec((1, UP), lambda i: (0, 0)),
            pl.BlockSpec((IN, 2 * UP), lambda i: (0, 0)),
            pl.BlockSpec((1, 2 * UP), lambda i: (0, 0)),
            pl.BlockSpec((IN, UP), lambda i: (0, 0)),
            pl.BlockSpec((1, UP), lambda i: (0, 0)),
        ],
        out_specs=[
            pl.BlockSpec((RB, UPW), lambda i: (i, 0)),
            pl.BlockSpec((RB, 2 * UP), lambda i: (i, 0)),
            pl.BlockSpec((RB, UP), lambda i: (i, 0)),
        ],
        out_shape=[
            jax.ShapeDtypeStruct((BN, UPW), jnp.float32),
            jax.ShapeDtypeStruct((BN, 2 * UP), jnp.float32),
            jax.ShapeDtypeStruct((BN, UP), jnp.float32),
        ],
    )(st2d, ip2d, Lst, Lip, lbp, Rst, Rip, rbp, W1ip, b1p, W2ip, b2p)


def _tc_post_body(x1_ref, p1_ref, p2_ref, w1h_ref, w2h_ref, bv_ref, out_ref):
    dot = functools.partial(jnp.dot, preferred_element_type=jnp.float32)
    st2 = x1_ref[...] + bv_ref[...]
    v = jax.nn.sigmoid(p1_ref[...] + dot(st2, w1h_ref[...]))
    r = v[:, :UP]
    u = v[:, UP:]
    c = jnp.tanh(p2_ref[...] + dot(r * st2, w2h_ref[...]))
    o = u * st2 + (1.0 - u) * c
    out_ref[...] = o[:, :U]


def _tc_post(x1, P1, P2, W1h, W2h, bvec):
    return pl.pallas_call(
        _tc_post_body,
        grid=(ROW_BLOCKS,),
        in_specs=[
            pl.BlockSpec((RB, UP), lambda i: (i, 0)),
            pl.BlockSpec((RB, 2 * UP), lambda i: (i, 0)),
            pl.BlockSpec((RB, UP), lambda i: (i, 0)),
            pl.BlockSpec((UP, 2 * UP), lambda i: (0, 0)),
            pl.BlockSpec((UP, UP), lambda i: (0, 0)),
            pl.BlockSpec((1, UP), lambda i: (0, 0)),
        ],
        out_specs=pl.BlockSpec((RB, U), lambda i: (i, 0)),
        out_shape=jax.ShapeDtypeStruct((BN, U), jnp.float32),
    )(x1, P1, P2, W1h, W2h, bvec)


_info = plsc.get_sparse_core_info()
_NC = _info.num_cores
_NS = _info.num_subcores
_NW = _NC * _NS
_BPW = B // _NW
_DEN = 320


def _sc_edge_body(xlr_hbm, src_hbm, dst_hbm, att_hbm, out_hbm,
                  xlr_v, out_v, src_v, dst_v, att_v, logit_v, e_v,
                  denom_v):
    cid = lax.axis_index("c")
    sid = lax.axis_index("s")
    wid = sid * _NC + cid
    pltpu.sync_copy(src_hbm, src_v)
    pltpu.sync_copy(dst_hbm, dst_v)
    pltpu.sync_copy(att_hbm, att_v)
    zero16 = jnp.zeros((16,), jnp.float32)

    for bi in range(_BPW):
        b = wid * _BPW + bi
        off = jnp.minimum((b * SPAN) // 8 * 8, XSZ - WIN)
        off = pl.multiple_of(off, 8)
        delta = b * SPAN - off
        pltpu.sync_copy(xlr_hbm.at[pl.ds(off, WIN)], xlr_v)

        def group_a(g, gmax):
            sbase = src_v[pl.ds(g * 16, 16)] * UPW + delta
            dbase = dst_v[pl.ds(g * 16, 16)] * UPW + (delta + XRO)

            def chunk_a(cu, accs):
                attc = att_v[pl.ds(cu * 16, 16)]
                bs = sbase + cu * 16
                bd = dbase + cu * 16
                outs = list(accs)
                for j in range(16):
                    xlc = plsc.load_gather(xlr_v, [bs + j])
                    xrc = plsc.load_gather(xlr_v, [bd + j])
                    m = xlc + xrc
                    m = jnp.maximum(m, 0.2 * m)
                    outs[j % 4] = outs[j % 4] + m * attc[j]
                return tuple(outs)

            a0, a1, a2, a3 = lax.fori_loop(0, UP // 16, chunk_a, (zero16,) * 4)
            acc = (a0 + a1) + (a2 + a3)
            lane = g * 16 + lax.iota(jnp.int32, 16)
            lg = jnp.where(lane < ET, acc, NEG)
            logit_v[pl.ds(g * 16, 16)] = lg
            return jnp.maximum(gmax, lg)

        gmaxv = plsc.parallel_loop(
            0, NG, unroll=2,
            carry=jnp.full((16,), NEG, jnp.float32))(group_a)
        gmax = jnp.max(gmaxv)

        for i in range(_DEN // 16):
            denom_v[pl.ds(i * 16, 16)] = zero16

        def group_b(g, carry):
            lg = logit_v[pl.ds(g * 16, 16)]
            e16 = jnp.exp(lg - gmax)
            e_v[pl.ds(g * 16, 16)] = e16
            dst16 = dst_v[pl.ds(g * 16, 16)]
            plsc.addupdate_scatter(denom_v, [dst16], e16)
            return carry

        plsc.parallel_loop(0, NG, unroll=2, carry=jnp.int32(0))(group_b)

        def zout(i, carry):
            for j in range(4):
                out_v[pl.ds((i * 4 + j) * 16, 16)] = zero16
            return carry

        plsc.parallel_loop(0, 544, unroll=2, carry=jnp.int32(0))(zout)

        def group_c(g, carry):
            dst16 = dst_v[pl.ds(g * 16, 16)]
            sbase = src_v[pl.ds(g * 16, 16)] * UPW + delta
            dbase = dst16 * UPS
            e16 = e_v[pl.ds(g * 16, 16)]
            den = plsc.load_gather(denom_v, [dst16])
            alpha = e16 / (den + 1e-16)

            def chunk_c(cu, carry2):
                bs = sbase + cu * 16
                bd = dbase + cu * 16
                for j in range(16):
                    xlc = plsc.load_gather(xlr_v, [bs + j])
                    plsc.addupdate_scatter(out_v, [bd + j], alpha * xlc)
                return carry2

            lax.fori_loop(0, UP // 16, chunk_c, 0)
            return carry

        plsc.parallel_loop(0, NG, unroll=2, carry=jnp.int32(0))(group_c)
        pltpu.sync_copy(out_v, out_hbm.at[b])


_sc_edge = functools.partial(
    pl.kernel,
    out_type=jax.ShapeDtypeStruct((B, NR * UPS), jnp.float32),
    mesh=plsc.VectorSubcoreMesh(core_axis_name="c", subcore_axis_name="s"),
    compiler_params=pltpu.CompilerParams(needs_layout_passes=False),
    scratch_types=[
        pltpu.VMEM((WIN,), jnp.float32),
        pltpu.VMEM((NR * UPS,), jnp.float32),
        pltpu.VMEM((ETP,), jnp.int32),
        pltpu.VMEM((ETP,), jnp.int32),
        pltpu.VMEM((UP + 16,), jnp.float32),
        pltpu.VMEM((ETP,), jnp.float32),
        pltpu.VMEM((ETP,), jnp.float32),
        pltpu.VMEM((_DEN,), jnp.float32),
    ],
)(_sc_edge_body)


def kernel(inputs, state, edge_index, bias_1, W_gcn1, b_gcn1, W_gcn2, b_gcn2,
           linl_w, linl_b, linr_w, linr_b, att, gat_bias):
    ip2d = inputs.reshape(BN, IN)
    st2d = state.reshape(BN, U)
    loops = jnp.arange(N, dtype=edge_index.dtype)
    src = jnp.pad(jnp.concatenate([edge_index[0], loops]), (0, ETP - ET))
    dst = jnp.pad(jnp.concatenate([edge_index[1], loops]), (0, ETP - ET))

    pad1 = lambda v: jnp.pad(v, (0, UP - U))
    row1 = lambda v: v.reshape(1, -1)
    Lst = jnp.pad(linl_w[:U], [(0, 0), (0, UP - U)])
    Lip = jnp.pad(linl_w[U:], [(0, 0), (0, UP - U)])
    Rst = jnp.pad(linr_w[:U], [(0, 0), (0, UP - U)])
    Rip = jnp.pad(linr_w[U:], [(0, 0), (0, UP - U)])
    W1h = jnp.concatenate(
        [jnp.pad(W_gcn1[IN:, :U], [(0, UP - U), (0, UP - U)]),
         jnp.pad(W_gcn1[IN:, U:], [(0, UP - U), (0, UP - U)])], axis=1)
    W1ip = jnp.concatenate(
        [jnp.pad(W_gcn1[:IN, :U], [(0, 0), (0, UP - U)]),
         jnp.pad(W_gcn1[:IN, U:], [(0, 0), (0, UP - U)])], axis=1)
    b1p = jnp.concatenate([pad1(b_gcn1[:U]), pad1(b_gcn1[U:])])
    W2h = jnp.pad(W_gcn2[IN:], [(0, UP - U), (0, UP - U)])
    W2ip = jnp.pad(W_gcn2[:IN], [(0, 0), (0, UP - U)])

    XLR, P1, P2 = _tc_pre(st2d, ip2d, Lst, Lip, row1(pad1(linl_b)),
                          Rst, Rip, row1(pad1(linr_b)),
                          W1ip, row1(b1p), W2ip, row1(pad1(b_gcn2)))

    x1 = _sc_edge(XLR.reshape(XSZ),
                  src.astype(jnp.int32), dst.astype(jnp.int32),
                  jnp.pad(att, (0, UP + 16 - U)))

    x1u = x1.reshape(B, NR, UPS)[:, :N, :UP].reshape(BN, UP)
    out = _tc_post(x1u, P1, P2, W1h, W2h,
                   row1(pad1(bias_1 + gat_bias)))
    return out.reshape(B, N * U)


# speedup vs baseline: 1.2536x; 1.0012x over previous
import functools

import jax
import jax.numpy as jnp
from jax import lax
from jax.experimental import pallas as pl
from jax.experimental.pallas import tpu as pltpu
from jax.experimental.pallas import tpu_sc as plsc

N = 307
U = 100
UP = 112
IN = 3
B = 64
E = 340
ET = E + N
NG = (ET + 15) // 16
ETP = NG * 16
BN = B * N
ROW_BLOCKS = 8
RB = BN // ROW_BLOCKS
NEG = -1e30
UPS = 113
NR = 312
UPW = 227
XRO = 112
SPAN = N * UPW
XSZ = BN * UPW
WIN = 69712


def _tc_pre_body(st_ref, ip_ref, lst_ref, lip_ref, lb_ref, rst_ref, rip_ref,
                 rb_ref, w1ip_ref, b1_ref, w2ip_ref, b2_ref,
                 xlr_ref, p1_ref, p2_ref):
    st = st_ref[...]
    ip = ip_ref[...]
    dot = functools.partial(jnp.dot, preferred_element_type=jnp.float32)
    xlr_ref[:, :UP] = dot(st, lst_ref[...]) + dot(ip, lip_ref[...]) + lb_ref[...]
    xlr_ref[:, XRO:XRO + UP] = (dot(st, rst_ref[...]) + dot(ip, rip_ref[...])
                                + rb_ref[...])
    p1_ref[...] = dot(ip, w1ip_ref[...]) + b1_ref[...]
    p2_ref[...] = dot(ip, w2ip_ref[...]) + b2_ref[...]


def _tc_pre(st2d, ip2d, Lst, Lip, lbp, Rst, Rip, rbp, W1ip, b1p, W2ip, b2p):
    return pl.pallas_call(
        _tc_pre_body,
        grid=(ROW_BLOCKS,),
        in_specs=[
            pl.BlockSpec((RB, U), lambda i: (i, 0)),
            pl.BlockSpec((RB, IN), lambda i: (i, 0)),
            pl.BlockSpec((U, UP), lambda i: (0, 0)),
            pl.BlockSpec((IN, UP), lambda i: (0, 0)),
            pl.BlockSpec((1, UP), lambda i: (0, 0)),
            pl.BlockSpec((U, UP), lambda i: (0, 0)),
            pl.BlockSpec((IN, UP), lambda i: (0, 0)),
            pl.BlockSpec((1, UP), lambda i: (0, 0)),
            pl.BlockSpec((IN, 2 * UP), lambda i: (0, 0)),
            pl.BlockSpec((1, 2 * UP), lambda i: (0, 0)),
            pl.BlockSpec((IN, UP), lambda i: (0, 0)),
            pl.BlockSpec((1, UP), lambda i: (0, 0)),
        ],
        out_specs=[
            pl.BlockSpec((RB, UPW), lambda i: (i, 0)),
            pl.BlockSpec((RB, 2 * UP), lambda i: (i, 0)),
            pl.BlockSpec((RB, UP), lambda i: (i, 0)),
        ],
        out_shape=[
            jax.ShapeDtypeStruct((BN, UPW), jnp.float32),
            jax.ShapeDtypeStruct((BN, 2 * UP), jnp.float32),
            jax.ShapeDtypeStruct((BN, UP), jnp.float32),
        ],
    )(st2d, ip2d, Lst, Lip, lbp, Rst, Rip, rbp, W1ip, b1p, W2ip, b2p)


def _tc_post_body(x1_ref, p1_ref, p2_ref, w1h_ref, w2h_ref, bv_ref, out_ref):
    dot = functools.partial(jnp.dot, preferred_element_type=jnp.float32)
    x1p = x1_ref[...]
    x1 = jnp.concatenate(
        [x1p[k * NR:k * NR + N, :UP] for k in range(B // ROW_BLOCKS)], axis=0)
    st2 = x1 + bv_ref[...]
    v = jax.nn.sigmoid(p1_ref[...] + dot(st2, w1h_ref[...]))
    r = v[:, :UP]
    u = v[:, UP:]
    c = jnp.tanh(p2_ref[...] + dot(r * st2, w2h_ref[...]))
    o = u * st2 + (1.0 - u) * c
    out_ref[...] = o[:, :U]


def _tc_post(x1, P1, P2, W1h, W2h, bvec):
    return pl.pallas_call(
        _tc_post_body,
        grid=(ROW_BLOCKS,),
        in_specs=[
            pl.BlockSpec((B // ROW_BLOCKS * NR, UPS), lambda i: (i, 0)),
            pl.BlockSpec((RB, 2 * UP), lambda i: (i, 0)),
            pl.BlockSpec((RB, UP), lambda i: (i, 0)),
            pl.BlockSpec((UP, 2 * UP), lambda i: (0, 0)),
            pl.BlockSpec((UP, UP), lambda i: (0, 0)),
            pl.BlockSpec((1, UP), lambda i: (0, 0)),
        ],
        out_specs=pl.BlockSpec((RB, U), lambda i: (i, 0)),
        out_shape=jax.ShapeDtypeStruct((BN, U), jnp.float32),
    )(x1, P1, P2, W1h, W2h, bvec)


_info = plsc.get_sparse_core_info()
_NC = _info.num_cores
_NS = _info.num_subcores
_NW = _NC * _NS
_BPW = B // _NW
_DEN = 320


def _sc_edge_body(xlr_hbm, src_hbm, dst_hbm, att_hbm, out_hbm,
                  xlr_v, out_v, src_v, dst_v, att_v, logit_v, e_v,
                  denom_v):
    cid = lax.axis_index("c")
    sid = lax.axis_index("s")
    wid = sid * _NC + cid
    pltpu.sync_copy(src_hbm, src_v)
    pltpu.sync_copy(dst_hbm, dst_v)
    pltpu.sync_copy(att_hbm, att_v)
    zero16 = jnp.zeros((16,), jnp.float32)

    for bi in range(_BPW):
        b = wid * _BPW + bi
        off = jnp.minimum((b * SPAN) // 8 * 8, XSZ - WIN)
        off = pl.multiple_of(off, 8)
        delta = b * SPAN - off
        pltpu.sync_copy(xlr_hbm.at[pl.ds(off, WIN)], xlr_v)

        def group_a(g, gmax):
            sbase = src_v[pl.ds(g * 16, 16)] * UPW + delta
            dbase = dst_v[pl.ds(g * 16, 16)] * UPW + (delta + XRO)

            def chunk_a(cu, accs):
                attc = att_v[pl.ds(cu * 16, 16)]
                bs = sbase + cu * 16
                bd = dbase + cu * 16
                outs = list(accs)
                for j in range(16):
                    xlc = plsc.load_gather(xlr_v, [bs + j])
                    xrc = plsc.load_gather(xlr_v, [bd + j])
                    m = xlc + xrc
                    m = jnp.maximum(m, 0.2 * m)
                    outs[j % 4] = outs[j % 4] + m * attc[j]
                return tuple(outs)

            a0, a1, a2, a3 = lax.fori_loop(0, UP // 16, chunk_a, (zero16,) * 4)
            acc = (a0 + a1) + (a2 + a3)
            lane = g * 16 + lax.iota(jnp.int32, 16)
            lg = jnp.where(lane < ET, acc, NEG)
            logit_v[pl.ds(g * 16, 16)] = lg
            return jnp.maximum(gmax, lg)

        gmaxv = plsc.parallel_loop(
            0, NG, unroll=2,
            carry=jnp.full((16,), NEG, jnp.float32))(group_a)
        gmax = jnp.max(gmaxv)

        for i in range(_DEN // 16):
            denom_v[pl.ds(i * 16, 16)] = zero16

        def group_b(g, carry):
            lg = logit_v[pl.ds(g * 16, 16)]
            e16 = jnp.exp(lg - gmax)
            e_v[pl.ds(g * 16, 16)] = e16
            dst16 = dst_v[pl.ds(g * 16, 16)]
            plsc.addupdate_scatter(denom_v, [dst16], e16)
            return carry

        plsc.parallel_loop(0, NG, unroll=2, carry=jnp.int32(0))(group_b)

        def zout(i, carry):
            for j in range(4):
                out_v[pl.ds((i * 4 + j) * 16, 16)] = zero16
            return carry

        plsc.parallel_loop(0, 544, unroll=2, carry=jnp.int32(0))(zout)

        def group_c(g, carry):
            dst16 = dst_v[pl.ds(g * 16, 16)]
            sbase = src_v[pl.ds(g * 16, 16)] * UPW + delta
            dbase = dst16 * UPS
            e16 = e_v[pl.ds(g * 16, 16)]
            den = plsc.load_gather(denom_v, [dst16])
            alpha = e16 / (den + 1e-16)

            def chunk_c(cu, carry2):
                bs = sbase + cu * 16
                bd = dbase + cu * 16
                for j in range(16):
                    xlc = plsc.load_gather(xlr_v, [bs + j])
                    plsc.addupdate_scatter(out_v, [bd + j], alpha * xlc)
                return carry2

            lax.fori_loop(0, UP // 16, chunk_c, 0)
            return carry

        plsc.parallel_loop(0, NG, unroll=2, carry=jnp.int32(0))(group_c)
        pltpu.sync_copy(
            out_v,
            out_hbm.at[pl.ds(pl.multiple_of(b * (NR * UPS), 8), NR * UPS)])


_sc_edge = functools.partial(
    pl.kernel,
    out_type=jax.ShapeDtypeStruct((B * NR * UPS,), jnp.float32),
    mesh=plsc.VectorSubcoreMesh(core_axis_name="c", subcore_axis_name="s"),
    compiler_params=pltpu.CompilerParams(needs_layout_passes=False),
    scratch_types=[
        pltpu.VMEM((WIN,), jnp.float32),
        pltpu.VMEM((NR * UPS,), jnp.float32),
        pltpu.VMEM((ETP,), jnp.int32),
        pltpu.VMEM((ETP,), jnp.int32),
        pltpu.VMEM((UP + 16,), jnp.float32),
        pltpu.VMEM((ETP,), jnp.float32),
        pltpu.VMEM((ETP,), jnp.float32),
        pltpu.VMEM((_DEN,), jnp.float32),
    ],
)(_sc_edge_body)


def kernel(inputs, state, edge_index, bias_1, W_gcn1, b_gcn1, W_gcn2, b_gcn2,
           linl_w, linl_b, linr_w, linr_b, att, gat_bias):
    ip2d = inputs.reshape(BN, IN)
    st2d = state.reshape(BN, U)
    loops = jnp.arange(N, dtype=edge_index.dtype)
    src = jnp.pad(jnp.concatenate([edge_index[0], loops]), (0, ETP - ET))
    dst = jnp.pad(jnp.concatenate([edge_index[1], loops]), (0, ETP - ET))

    pad1 = lambda v: jnp.pad(v, (0, UP - U))
    row1 = lambda v: v.reshape(1, -1)
    Lst = jnp.pad(linl_w[:U], [(0, 0), (0, UP - U)])
    Lip = jnp.pad(linl_w[U:], [(0, 0), (0, UP - U)])
    Rst = jnp.pad(linr_w[:U], [(0, 0), (0, UP - U)])
    Rip = jnp.pad(linr_w[U:], [(0, 0), (0, UP - U)])
    W1h = jnp.concatenate(
        [jnp.pad(W_gcn1[IN:, :U], [(0, UP - U), (0, UP - U)]),
         jnp.pad(W_gcn1[IN:, U:], [(0, UP - U), (0, UP - U)])], axis=1)
    W1ip = jnp.concatenate(
        [jnp.pad(W_gcn1[:IN, :U], [(0, 0), (0, UP - U)]),
         jnp.pad(W_gcn1[:IN, U:], [(0, 0), (0, UP - U)])], axis=1)
    b1p = jnp.concatenate([pad1(b_gcn1[:U]), pad1(b_gcn1[U:])])
    W2h = jnp.pad(W_gcn2[IN:], [(0, UP - U), (0, UP - U)])
    W2ip = jnp.pad(W_gcn2[:IN], [(0, 0), (0, UP - U)])

    XLR, P1, P2 = _tc_pre(st2d, ip2d, Lst, Lip, row1(pad1(linl_b)),
                          Rst, Rip, row1(pad1(linr_b)),
                          W1ip, row1(b1p), W2ip, row1(pad1(b_gcn2)))

    x1 = _sc_edge(XLR.reshape(XSZ),
                  src.astype(jnp.int32), dst.astype(jnp.int32),
                  jnp.pad(att, (0, UP + 16 - U)))

    out = _tc_post(x1.reshape(B * NR, UPS), P1, P2, W1h, W2h,
                   row1(pad1(bias_1 + gat_bias)))
    return out.reshape(B, N * U)


# speedup vs baseline: 1.4714x; 1.1737x over previous
import functools

import jax
import jax.numpy as jnp
from jax import lax
from jax.experimental import pallas as pl
from jax.experimental.pallas import tpu as pltpu
from jax.experimental.pallas import tpu_sc as plsc

N = 307
U = 100
UP = 112
IN = 3
B = 64
E = 340
ET = E + N
NG = (ET + 15) // 16
ETP = NG * 16
BN = B * N
ROW_BLOCKS = 4
RB = BN // ROW_BLOCKS
NEG = -1e30
UPS = 113
NR = 312
UPW = 227
XRO = 112
SPAN = N * UPW
XSZ = BN * UPW
WIN = 69712


def _tc_pre_body(st_ref, ip_ref, lst_ref, lip_ref, lb_ref, rst_ref, rip_ref,
                 rb_ref, w1ip_ref, b1_ref, w2ip_ref, b2_ref,
                 xlr_ref, p1_ref, p2_ref):
    st = st_ref[...]
    ip = ip_ref[...]
    dot = functools.partial(jnp.dot, preferred_element_type=jnp.float32)
    xlr_ref[:, :UP] = dot(st, lst_ref[...]) + dot(ip, lip_ref[...]) + lb_ref[...]
    xlr_ref[:, XRO:XRO + UP] = (dot(st, rst_ref[...]) + dot(ip, rip_ref[...])
                                + rb_ref[...])
    p1_ref[...] = dot(ip, w1ip_ref[...]) + b1_ref[...]
    p2_ref[...] = dot(ip, w2ip_ref[...]) + b2_ref[...]


def _tc_pre(st2d, ip2d, Lst, Lip, lbp, Rst, Rip, rbp, W1ip, b1p, W2ip, b2p):
    return pl.pallas_call(
        _tc_pre_body,
        grid=(ROW_BLOCKS,),
        in_specs=[
            pl.BlockSpec((RB, U), lambda i: (i, 0)),
            pl.BlockSpec((RB, IN), lambda i: (i, 0)),
            pl.BlockSpec((U, UP), lambda i: (0, 0)),
            pl.BlockSpec((IN, UP), lambda i: (0, 0)),
            pl.BlockSpec((1, UP), lambda i: (0, 0)),
            pl.BlockSpec((U, UP), lambda i: (0, 0)),
            pl.BlockSpec((IN, UP), lambda i: (0, 0)),
            pl.BlockSpec((1, UP), lambda i: (0, 0)),
            pl.BlockSpec((IN, 2 * UP), lambda i: (0, 0)),
            pl.BlockSpec((1, 2 * UP), lambda i: (0, 0)),
            pl.BlockSpec((IN, UP), lambda i: (0, 0)),
            pl.BlockSpec((1, UP), lambda i: (0, 0)),
        ],
        out_specs=[
            pl.BlockSpec((RB, UPW), lambda i: (i, 0)),
            pl.BlockSpec((RB, 2 * UP), lambda i: (i, 0)),
            pl.BlockSpec((RB, UP), lambda i: (i, 0)),
        ],
        out_shape=[
            jax.ShapeDtypeStruct((BN, UPW), jnp.float32),
            jax.ShapeDtypeStruct((BN, 2 * UP), jnp.float32),
            jax.ShapeDtypeStruct((BN, UP), jnp.float32),
        ],
    )(st2d, ip2d, Lst, Lip, lbp, Rst, Rip, rbp, W1ip, b1p, W2ip, b2p)


def _tc_post_body(x1_ref, p1_ref, p2_ref, w1h_ref, w2h_ref, bv_ref, out_ref):
    dot = functools.partial(jnp.dot, preferred_element_type=jnp.float32)
    x1p = x1_ref[...]
    x1 = jnp.concatenate(
        [x1p[k * NR:k * NR + N, :UP] for k in range(B // ROW_BLOCKS)], axis=0)
    st2 = x1 + bv_ref[...]
    v = jax.nn.sigmoid(p1_ref[...] + dot(st2, w1h_ref[...]))
    r = v[:, :UP]
    u = v[:, UP:]
    c = jnp.tanh(p2_ref[...] + dot(r * st2, w2h_ref[...]))
    o = u * st2 + (1.0 - u) * c
    out_ref[...] = o[:, :U]


def _tc_post(x1, P1, P2, W1h, W2h, bvec):
    return pl.pallas_call(
        _tc_post_body,
        grid=(ROW_BLOCKS,),
        in_specs=[
            pl.BlockSpec((B // ROW_BLOCKS * NR, UPS), lambda i: (i, 0)),
            pl.BlockSpec((RB, 2 * UP), lambda i: (i, 0)),
            pl.BlockSpec((RB, UP), lambda i: (i, 0)),
            pl.BlockSpec((UP, 2 * UP), lambda i: (0, 0)),
            pl.BlockSpec((UP, UP), lambda i: (0, 0)),
            pl.BlockSpec((1, UP), lambda i: (0, 0)),
        ],
        out_specs=pl.BlockSpec((RB, U), lambda i: (i, 0)),
        out_shape=jax.ShapeDtypeStruct((BN, U), jnp.float32),
    )(x1, P1, P2, W1h, W2h, bvec)


_info = plsc.get_sparse_core_info()
_NC = _info.num_cores
_NS = _info.num_subcores
_NW = _NC * _NS
_BPW = B // _NW
_DEN = 320


def _sc_edge_body(xlr_hbm, src_hbm, dst_hbm, att_hbm, out_hbm,
                  xlr_v, out_v, src_v, dst_v, att_v, logit_v, e_v,
                  denom_v):
    cid = lax.axis_index("c")
    sid = lax.axis_index("s")
    wid = sid * _NC + cid
    pltpu.sync_copy(src_hbm, src_v)
    pltpu.sync_copy(dst_hbm, dst_v)
    pltpu.sync_copy(att_hbm, att_v)
    zero16 = jnp.zeros((16,), jnp.float32)

    for bi in range(_BPW):
        b = wid * _BPW + bi
        off = jnp.minimum((b * SPAN) // 8 * 8, XSZ - WIN)
        off = pl.multiple_of(off, 8)
        delta = b * SPAN - off
        pltpu.sync_copy(xlr_hbm.at[pl.ds(off, WIN)], xlr_v)

        def group_a(g, gmax):
            sbase = src_v[pl.ds(g * 16, 16)] * UPW + delta
            dbase = dst_v[pl.ds(g * 16, 16)] * UPW + (delta + XRO)

            def chunk_a(cu, accs):
                attc = att_v[pl.ds(cu * 16, 16)]
                bs = sbase + cu * 16
                bd = dbase + cu * 16
                outs = list(accs)
                for j in range(16):
                    xlc = plsc.load_gather(xlr_v, [bs + j])
                    xrc = plsc.load_gather(xlr_v, [bd + j])
                    m = xlc + xrc
                    m = jnp.maximum(m, 0.2 * m)
                    outs[j % 4] = outs[j % 4] + m * attc[j]
                return tuple(outs)

            a0, a1, a2, a3 = plsc.parallel_loop(
                0, UP // 16, carry=(zero16,) * 4)(chunk_a)
            acc = (a0 + a1) + (a2 + a3)
            lane = g * 16 + lax.iota(jnp.int32, 16)
            lg = jnp.where(lane < ET, acc, NEG)
            logit_v[pl.ds(g * 16, 16)] = lg
            return jnp.maximum(gmax, lg)

        gmaxv = plsc.parallel_loop(
            0, NG, unroll=2,
            carry=jnp.full((16,), NEG, jnp.float32))(group_a)
        gmax = jnp.max(gmaxv)

        for i in range(_DEN // 16):
            denom_v[pl.ds(i * 16, 16)] = zero16

        def group_b(g, carry):
            lg = logit_v[pl.ds(g * 16, 16)]
            e16 = jnp.exp(lg - gmax)
            e_v[pl.ds(g * 16, 16)] = e16
            dst16 = dst_v[pl.ds(g * 16, 16)]
            plsc.addupdate_scatter(denom_v, [dst16], e16)
            return carry

        plsc.parallel_loop(0, NG, unroll=2, carry=jnp.int32(0))(group_b)

        def zout(i, carry):
            for j in range(4):
                out_v[pl.ds((i * 4 + j) * 16, 16)] = zero16
            return carry

        plsc.parallel_loop(0, 544, unroll=2, carry=jnp.int32(0))(zout)

        def group_c(g, carry):
            dst16 = dst_v[pl.ds(g * 16, 16)]
            sbase = src_v[pl.ds(g * 16, 16)] * UPW + delta
            dbase = dst16 * UPS
            e16 = e_v[pl.ds(g * 16, 16)]
            den = plsc.load_gather(denom_v, [dst16])
            alpha = e16 / (den + 1e-16)

            def chunk_c(cu, carry2):
                bs = sbase + cu * 16
                bd = dbase + cu * 16
                for j in range(16):
                    xlc = plsc.load_gather(xlr_v, [bs + j])
                    plsc.addupdate_scatter(out_v, [bd + j], alpha * xlc)
                return carry2

            plsc.parallel_loop(0, UP // 16, carry=jnp.int32(0))(chunk_c)
            return carry

        plsc.parallel_loop(0, NG, unroll=2, carry=jnp.int32(0))(group_c)
        pltpu.sync_copy(
            out_v,
            out_hbm.at[pl.ds(pl.multiple_of(b * (NR * UPS), 8), NR * UPS)])


_sc_edge = functools.partial(
    pl.kernel,
    out_type=jax.ShapeDtypeStruct((B * NR * UPS,), jnp.float32),
    mesh=plsc.VectorSubcoreMesh(core_axis_name="c", subcore_axis_name="s"),
    compiler_params=pltpu.CompilerParams(needs_layout_passes=False),
    scratch_types=[
        pltpu.VMEM((WIN,), jnp.float32),
        pltpu.VMEM((NR * UPS,), jnp.float32),
        pltpu.VMEM((ETP,), jnp.int32),
        pltpu.VMEM((ETP,), jnp.int32),
        pltpu.VMEM((UP + 16,), jnp.float32),
        pltpu.VMEM((ETP,), jnp.float32),
        pltpu.VMEM((ETP,), jnp.float32),
        pltpu.VMEM((_DEN,), jnp.float32),
    ],
)(_sc_edge_body)


def kernel(inputs, state, edge_index, bias_1, W_gcn1, b_gcn1, W_gcn2, b_gcn2,
           linl_w, linl_b, linr_w, linr_b, att, gat_bias):
    ip2d = inputs.reshape(BN, IN)
    st2d = state.reshape(BN, U)
    loops = jnp.arange(N, dtype=edge_index.dtype)
    src = jnp.pad(jnp.concatenate([edge_index[0], loops]), (0, ETP - ET))
    dst = jnp.pad(jnp.concatenate([edge_index[1], loops]), (0, ETP - ET))

    pad1 = lambda v: jnp.pad(v, (0, UP - U))
    row1 = lambda v: v.reshape(1, -1)
    Lst = jnp.pad(linl_w[:U], [(0, 0), (0, UP - U)])
    Lip = jnp.pad(linl_w[U:], [(0, 0), (0, UP - U)])
    Rst = jnp.pad(linr_w[:U], [(0, 0), (0, UP - U)])
    Rip = jnp.pad(linr_w[U:], [(0, 0), (0, UP - U)])
    W1h = jnp.concatenate(
        [jnp.pad(W_gcn1[IN:, :U], [(0, UP - U), (0, UP - U)]),
         jnp.pad(W_gcn1[IN:, U:], [(0, UP - U), (0, UP - U)])], axis=1)
    W1ip = jnp.concatenate(
        [jnp.pad(W_gcn1[:IN, :U], [(0, 0), (0, UP - U)]),
         jnp.pad(W_gcn1[:IN, U:], [(0, 0), (0, UP - U)])], axis=1)
    b1p = jnp.concatenate([pad1(b_gcn1[:U]), pad1(b_gcn1[U:])])
    W2h = jnp.pad(W_gcn2[IN:], [(0, UP - U), (0, UP - U)])
    W2ip = jnp.pad(W_gcn2[:IN], [(0, 0), (0, UP - U)])

    XLR, P1, P2 = _tc_pre(st2d, ip2d, Lst, Lip, row1(pad1(linl_b)),
                          Rst, Rip, row1(pad1(linr_b)),
                          W1ip, row1(b1p), W2ip, row1(pad1(b_gcn2)))

    x1 = _sc_edge(XLR.reshape(XSZ),
                  src.astype(jnp.int32), dst.astype(jnp.int32),
                  jnp.pad(att, (0, UP + 16 - U)))

    out = _tc_post(x1.reshape(B * NR, UPS), P1, P2, W1h, W2h,
                   row1(pad1(bias_1 + gat_bias)))
    return out.reshape(B, N * U)
